# Initial kernel scaffold; baseline (speedup 1.0000x reference)
import jax
import jax.numpy as jnp
from jax.experimental import pallas as pl


def kernel(inps, unary_tensor, W_emb, b_emb, W_gcn, b_gcn, W1, b1, W2, b2):
    raise NotImplementedError("write your pallas kernel here")


# speedup vs baseline: 19.6234x; 19.6234x over previous
import jax
import jax.numpy as jnp
from jax.experimental import pallas as pl

_NB = 16
_BS = 64
_HID = 128
_NACT = 16
_FEAT = 3


def _fused_kernel(x_ref, wt_ref, be_ref, wg_ref, bg_ref, w1_ref, b1_ref,
                  w2_ref, b2_ref, act_ref, out_ref):
    xm = jnp.dot(x_ref[...], wt_ref[...],
                 preferred_element_type=jnp.float32) * (1.0 / _NB) + be_ref[...]
    g = jnp.dot(xm, wg_ref[...], preferred_element_type=jnp.float32) + bg_ref[...]
    h = jnp.dot(g, w1_ref[...], preferred_element_type=jnp.float32) + b1_ref[...]
    h = jnp.where(h >= 0, h, 0.01 * h)
    qa = jnp.dot(h, w2_ref[...], preferred_element_type=jnp.float32) + b2_ref[...]
    act = act_ref[...]
    amax = jnp.max(act, axis=1, keepdims=True)
    col = jax.lax.broadcasted_iota(jnp.int32, (_BS, _NACT), 1)
    idx = jnp.min(jnp.where(act == amax, col, _NACT), axis=1, keepdims=True)
    out_ref[...] = jnp.sum(jnp.where(col == idx, qa, 0.0), axis=1, keepdims=True)


def kernel(inps, unary_tensor, W_emb, b_emb, W_gcn, b_gcn, W1, b1, W2, b2):
    actions = inps[0, 1]
    x = unary_tensor.reshape(_BS, _NB * _FEAT)
    wt = jnp.tile(W_emb.T, (_NB, 1))
    return pl.pallas_call(
        _fused_kernel,
        out_shape=jax.ShapeDtypeStruct((_BS, 1), jnp.float32),
    )(x, wt, b_emb.reshape(1, _HID), W_gcn.T, b_gcn.reshape(1, _HID),
      W1.T, b1.reshape(1, _HID), W2.T, b2.reshape(1, _NACT), actions)
